# baseline (speedup 1.0000x reference)
import functools

import jax
import jax.numpy as jnp
from jax import lax
from jax.experimental import pallas as pl
from jax.experimental.pallas import tpu as pltpu
from jax.experimental.pallas import tpu_sc as plsc

N_PAD = 10240
BLK = 1024
NC = 2
NS = 16
LANES = 16

_HIGH = jax.lax.Precision.DEFAULT


def _tc1_body(x_ref, wp_ref, bp_ref, out_ref):
    acc = lax.dot_general(wp_ref[...], x_ref[...], (((0,), (1,)), ((), ())),
                          precision=_HIGH, preferred_element_type=jnp.float32)
    out_ref[...] = jnp.maximum(acc + bp_ref[...], 0.0)


def _tc2_body(x_ref, hn_ref, ws_ref, bs_ref, wn_ref, wp2_ref, bp2_ref,
              h_ref, hp2_ref):
    rst = lax.dot_general(ws_ref[...], x_ref[...], (((0,), (1,)), ((), ())),
                          precision=_HIGH, preferred_element_type=jnp.float32)
    rst = rst + bs_ref[...]
    rst = rst + lax.dot_general(wn_ref[...], hn_ref[...],
                                (((0,), (0,)), ((), ())),
                                precision=_HIGH,
                                preferred_element_type=jnp.float32)
    h = jnp.maximum(rst, 0.0)
    h_ref[...] = h
    hp2 = lax.dot_general(wp2_ref[...], h, (((0,), (0,)), ((), ())),
                          precision=_HIGH, preferred_element_type=jnp.float32)
    hp2_ref[...] = jnp.maximum(hp2 + bp2_ref[...], 0.0)


def _tc3_body(h_ref, hn2_ref, ws2_ref, bs2_ref, wn2_ref, wo_ref, bo_ref,
              out_ref):
    rst = lax.dot_general(ws2_ref[...], h_ref[...], (((0,), (0,)), ((), ())),
                          precision=_HIGH, preferred_element_type=jnp.float32)
    rst = rst + bs2_ref[...]
    rst = rst + lax.dot_general(wn2_ref[...], hn2_ref[...],
                                (((0,), (0,)), ((), ())),
                                precision=_HIGH,
                                preferred_element_type=jnp.float32)
    h2 = jnp.maximum(rst, 0.0)
    out = lax.dot_general(wo_ref[...], h2, (((0,), (0,)), ((), ())),
                          precision=_HIGH, preferred_element_type=jnp.float32)
    out_ref[...] = out + bo_ref[...]


def _full(shape):
    return pl.BlockSpec(shape, lambda i: (0, 0))


def _make_segmax(f_total, f_tile, e_pad, ec, ng, nbanks, packed):
    nchunks = e_pad // ec
    mesh = plsc.VectorSubcoreMesh(core_axis_name="c", subcore_axis_name="s",
                                  num_cores=NC, num_subcores=NS)

    scratch = (
        [pltpu.VMEM((1, N_PAD), jnp.float32)] * f_tile
        + [pltpu.VMEM((1, N_PAD), jnp.float32)] * (f_tile * nbanks)
        + [pltpu.VMEM((ec,), jnp.int32)] * (2 if packed else 4)
        + [pltpu.SMEM((ec // LANES,), jnp.int32),
           pltpu.SemaphoreType.DMA, pltpu.SemaphoreType.DMA]
    )

    @functools.partial(
        pl.kernel,
        out_type=jax.ShapeDtypeStruct((f_total, N_PAD), jnp.float32),
        mesh=mesh,
        compiler_params=pltpu.CompilerParams(needs_layout_passes=False),
        scratch_types=scratch,
    )
    def seg(hpT_hbm, *args):
        if packed:
            (sd_hbm, out_hbm), scr = args[:2], args[2:]
        else:
            (src_hbm, dst_hbm, out_hbm), scr = args[:3], args[3:]
        hps = scr[:f_tile]
        accs = scr[f_tile:f_tile + f_tile * nbanks]
        rest = scr[f_tile + f_tile * nbanks:]
        if packed:
            e_bufs = ((rest[0],), (rest[1],))
            glist_s, sem0, sem1 = rest[2:]
        else:
            e_bufs = ((rest[0], rest[2]), (rest[1], rest[3]))
            glist_s, sem0, sem1 = rest[4:]
        sems = (sem0, sem1)

        def acc_ref(a, f):
            return accs[(a % nbanks) * f_tile + f]

        wid = lax.axis_index("s") * NC + lax.axis_index("c")
        fb = wid * f_tile

        def start_chunk(slot, cidx):
            if packed:
                pltpu.async_copy(sd_hbm.at[pl.ds(cidx * ec, ec)],
                                 e_bufs[slot][0], sems[slot])
            else:
                pltpu.async_copy(src_hbm.at[pl.ds(cidx * ec, ec)],
                                 e_bufs[slot][0], sems[slot])
                pltpu.async_copy(dst_hbm.at[pl.ds(cidx * ec, ec)],
                                 e_bufs[slot][1], sems[slot])

        start_chunk(0, 0)
        if nchunks > 1:
            start_chunk(1, 1)

        for f in range(f_tile):
            pltpu.sync_copy(hpT_hbm.at[pl.ds(fb + f, 1)], hps[f])

        def zero_body(i, carry):
            for a_v in accs:
                a_v[0, pl.ds(i * LANES, LANES)] = jnp.zeros((LANES,),
                                                            jnp.float32)
            return carry
        lax.fori_loop(0, N_PAD // LANES, zero_body, 0)

        zrow = jnp.zeros((LANES,), jnp.int32)

        def wait_chunk(slot):
            src0 = sd_hbm if packed else src_hbm
            for b in e_bufs[slot]:
                pltpu.make_async_copy(src0.at[pl.ds(0, ec)], b,
                                      sems[slot]).wait()

        def load_sd(bufs, g):
            if packed:
                sd = bufs[0][pl.ds(g * LANES, LANES)]
                return (jnp.bitwise_and(sd, jnp.int32(0x3FFF)),
                        lax.shift_right_logical(sd, jnp.int32(14)))
            return (bufs[0][pl.ds(g * LANES, LANES)],
                    bufs[1][pl.ds(g * LANES, LANES)])

        def process_chunk(bufs):
            def group_body(it, cnt):
                pairs = [load_sd(bufs, it * ng + a) for a in range(ng)]
                ss_ = [p[0] for p in pairs]
                ds_ = [p[1] for p in pairs]
                msgs = [[plsc.load_gather(hps[f], [zrow, ss_[a]])
                         for f in range(f_tile)] for a in range(ng)]
                curs = [[plsc.load_gather(acc_ref(a, f), [zrow, ds_[a]])
                         for f in range(f_tile)] for a in range(ng)]
                for a in range(ng):
                    for f in range(f_tile):
                        plsc.store_scatter(
                            acc_ref(a, f), [zrow, ds_[a]], msgs[a][f],
                            mask=msgs[a][f] > curs[a][f])
                chks = [[plsc.load_gather(acc_ref(a, f), [zrow, ds_[a]])
                         for f in range(f_tile)] for a in range(ng)]
                loses = []
                for a in range(ng):
                    lose = msgs[a][0] > chks[a][0]
                    for f in range(1, f_tile):
                        lose = jnp.logical_or(lose,
                                              msgs[a][f] > chks[a][f])
                    loses.append(lose)
                for a in range(ng):
                    nlose = plsc.all_reduce_population_count(loses[a])[0]
                    glist_s[cnt] = it * ng + a
                    cnt = cnt + (nlose > 0).astype(jnp.int32)
                return cnt
            ndup = lax.fori_loop(0, ec // (LANES * ng), group_body,
                                 jnp.int32(0))

            def dup_body(i, carry):
                g = glist_s[i]
                s, d = load_sd(bufs, g)
                for f in range(f_tile):
                    msg = plsc.load_gather(hps[f], [zrow, s])

                    def cond(cur2):
                        return jnp.any(msg > cur2)

                    def body(cur2):
                        plsc.store_scatter(accs[f], [zrow, d], msg,
                                           mask=msg > cur2)
                        return plsc.load_gather(accs[f], [zrow, d])
                    lax.while_loop(cond, body,
                                   plsc.load_gather(accs[f], [zrow, d]))
                return carry
            lax.fori_loop(0, ndup, dup_body, 0)

        def pair_body(k, carry):
            for slot in range(2):
                c = 2 * k + slot
                wait_chunk(slot)
                process_chunk(e_bufs[slot])

                @pl.when(c + 2 < nchunks)
                def _start_next(slot=slot, c=c):
                    start_chunk(slot, c + 2)
            return carry
        lax.fori_loop(0, nchunks // 2, pair_body, 0)

        if nbanks > 1:
            def merge_body(i, carry):
                for f in range(f_tile):
                    m = accs[f][0, pl.ds(i * LANES, LANES)]
                    for a in range(1, nbanks):
                        m = jnp.maximum(
                            m, accs[a * f_tile + f][0,
                                                    pl.ds(i * LANES, LANES)])
                    accs[f][0, pl.ds(i * LANES, LANES)] = m
                return carry
            lax.fori_loop(0, N_PAD // LANES, merge_body, 0)

        for f in range(f_tile):
            pltpu.sync_copy(accs[f], out_hbm.at[pl.ds(fb + f, 1)])

    return seg


def kernel(x, edge_index, W_pool1, b_pool1, W_self1, b_self1, W_neigh1,
           W_pool2, b_pool2, W_self2, b_self2, W_neigh2, W_out, b_out):
    n, d_in = x.shape
    h_dim = W_self1.shape[1]
    c_dim = W_out.shape[1]
    e = edge_index.shape[1]

    ec = 8000
    e_pad = ((e + 2 * ec - 1) // (2 * ec)) * (2 * ec)
    src = edge_index[0]
    dst = edge_index[1]
    if e_pad != e:
        src = jnp.concatenate([src, jnp.zeros((e_pad - e,), jnp.int32)])
        dst = jnp.concatenate(
            [dst, jnp.full((e_pad - e,), n, jnp.int32)])
    sd_packed = jnp.bitwise_or(src, jnp.left_shift(dst, 14))

    xp = jnp.pad(x, ((0, N_PAD - n), (0, 0)))
    grid = N_PAD // BLK

    h_pool1T = pl.pallas_call(
        _tc1_body,
        grid=(grid,),
        in_specs=[pl.BlockSpec((BLK, d_in), lambda i: (i, 0)),
                  _full((d_in, d_in)),
                  _full((d_in, 1))],
        out_specs=pl.BlockSpec((d_in, BLK), lambda i: (0, i)),
        out_shape=jax.ShapeDtypeStruct((d_in, N_PAD), jnp.float32),
    )(xp, W_pool1, b_pool1.reshape(d_in, 1))

    seg1 = _make_segmax(d_in, d_in // (NC * NS), e_pad, ec, ng=5, nbanks=1,
                        packed=True)
    h_neigh1T = seg1(h_pool1T, sd_packed)

    hT, h_pool2T = pl.pallas_call(
        _tc2_body,
        grid=(grid,),
        in_specs=[pl.BlockSpec((BLK, d_in), lambda i: (i, 0)),
                  pl.BlockSpec((d_in, BLK), lambda i: (0, i)),
                  _full((d_in, h_dim)),
                  _full((h_dim, 1)),
                  _full((d_in, h_dim)),
                  _full((h_dim, h_dim)),
                  _full((h_dim, 1))],
        out_specs=[pl.BlockSpec((h_dim, BLK), lambda i: (0, i)),
                   pl.BlockSpec((h_dim, BLK), lambda i: (0, i))],
        out_shape=[jax.ShapeDtypeStruct((h_dim, N_PAD), jnp.float32),
                   jax.ShapeDtypeStruct((h_dim, N_PAD), jnp.float32)],
    )(xp, h_neigh1T, W_self1, b_self1.reshape(h_dim, 1), W_neigh1,
      W_pool2, b_pool2.reshape(h_dim, 1))

    seg2 = _make_segmax(h_dim, h_dim // (NC * NS), e_pad, ec, ng=10,
                        nbanks=10, packed=True)
    h_neigh2T = seg2(h_pool2T, sd_packed)

    outT = pl.pallas_call(
        _tc3_body,
        grid=(grid,),
        in_specs=[pl.BlockSpec((h_dim, BLK), lambda i: (0, i)),
                  pl.BlockSpec((h_dim, BLK), lambda i: (0, i)),
                  _full((h_dim, h_dim)),
                  _full((h_dim, 1)),
                  _full((h_dim, h_dim)),
                  _full((h_dim, c_dim)),
                  _full((c_dim, 1))],
        out_specs=pl.BlockSpec((c_dim, BLK), lambda i: (0, i)),
        out_shape=jax.ShapeDtypeStruct((c_dim, N_PAD), jnp.float32),
    )(hT, h_neigh2T, W_self2, b_self2.reshape(h_dim, 1), W_neigh2,
      W_out, b_out.reshape(c_dim, 1))

    return outT[:, :n].T


# speedup vs baseline: 1.2014x; 1.2014x over previous
import functools

import jax
import jax.numpy as jnp
from jax import lax
from jax.experimental import pallas as pl
from jax.experimental.pallas import tpu as pltpu
from jax.experimental.pallas import tpu_sc as plsc

N_PAD = 10240
BLK = 1024
NC = 2
NS = 16
LANES = 16

_HIGH = jax.lax.Precision.DEFAULT


def _tc1_body(x_ref, wp_ref, bp_ref, out_ref):
    acc = lax.dot_general(wp_ref[...], x_ref[...], (((0,), (1,)), ((), ())),
                          precision=_HIGH, preferred_element_type=jnp.float32)
    out_ref[...] = jnp.maximum(acc + bp_ref[...], 0.0)


def _tc2_body(x_ref, hn_ref, ws_ref, bs_ref, wn_ref, wp2_ref, bp2_ref,
              h_ref, hp2_ref):
    rst = lax.dot_general(ws_ref[...], x_ref[...], (((0,), (1,)), ((), ())),
                          precision=_HIGH, preferred_element_type=jnp.float32)
    rst = rst + bs_ref[...]
    rst = rst + lax.dot_general(wn_ref[...], hn_ref[...],
                                (((0,), (0,)), ((), ())),
                                precision=_HIGH,
                                preferred_element_type=jnp.float32)
    h = jnp.maximum(rst, 0.0)
    h_ref[...] = h
    hp2 = lax.dot_general(wp2_ref[...], h, (((0,), (0,)), ((), ())),
                          precision=_HIGH, preferred_element_type=jnp.float32)
    hp2_ref[...] = jnp.maximum(hp2 + bp2_ref[...], 0.0)


def _tc3_body(h_ref, hn2_ref, ws2_ref, bs2_ref, wn2_ref, wo_ref, bo_ref,
              out_ref):
    rst = lax.dot_general(ws2_ref[...], h_ref[...], (((0,), (0,)), ((), ())),
                          precision=_HIGH, preferred_element_type=jnp.float32)
    rst = rst + bs2_ref[...]
    rst = rst + lax.dot_general(wn2_ref[...], hn2_ref[...],
                                (((0,), (0,)), ((), ())),
                                precision=_HIGH,
                                preferred_element_type=jnp.float32)
    h2 = jnp.maximum(rst, 0.0)
    out = lax.dot_general(wo_ref[...], h2, (((0,), (0,)), ((), ())),
                          precision=_HIGH, preferred_element_type=jnp.float32)
    out_ref[...] = out + bo_ref[...]


def _full(shape):
    return pl.BlockSpec(shape, lambda i: (0, 0))


def _make_segmax(f_total, f_tile, e_pad, ec, ng, nbanks, packed):
    nchunks = e_pad // ec
    mesh = plsc.VectorSubcoreMesh(core_axis_name="c", subcore_axis_name="s",
                                  num_cores=NC, num_subcores=NS)

    scratch = (
        [pltpu.VMEM((1, N_PAD), jnp.float32)] * f_tile
        + [pltpu.VMEM((1, N_PAD), jnp.float32)] * (f_tile * nbanks)
        + [pltpu.VMEM((ec,), jnp.int32)] * (2 if packed else 4)
        + [pltpu.SMEM((ec // LANES,), jnp.int32),
           pltpu.SemaphoreType.DMA, pltpu.SemaphoreType.DMA]
    )

    @functools.partial(
        pl.kernel,
        out_type=jax.ShapeDtypeStruct((f_total, N_PAD), jnp.float32),
        mesh=mesh,
        compiler_params=pltpu.CompilerParams(needs_layout_passes=False),
        scratch_types=scratch,
    )
    def seg(hpT_hbm, *args):
        if packed:
            (sd_hbm, out_hbm), scr = args[:2], args[2:]
        else:
            (src_hbm, dst_hbm, out_hbm), scr = args[:3], args[3:]
        hps = scr[:f_tile]
        accs = scr[f_tile:f_tile + f_tile * nbanks]
        rest = scr[f_tile + f_tile * nbanks:]
        if packed:
            e_bufs = ((rest[0],), (rest[1],))
            glist_s, sem0, sem1 = rest[2:]
        else:
            e_bufs = ((rest[0], rest[2]), (rest[1], rest[3]))
            glist_s, sem0, sem1 = rest[4:]
        sems = (sem0, sem1)

        def acc_ref(a, f):
            return accs[(a % nbanks) * f_tile + f]

        wid = lax.axis_index("s") * NC + lax.axis_index("c")
        fb = wid * f_tile

        def start_chunk(slot, cidx):
            if packed:
                pltpu.async_copy(sd_hbm.at[pl.ds(cidx * ec, ec)],
                                 e_bufs[slot][0], sems[slot])
            else:
                pltpu.async_copy(src_hbm.at[pl.ds(cidx * ec, ec)],
                                 e_bufs[slot][0], sems[slot])
                pltpu.async_copy(dst_hbm.at[pl.ds(cidx * ec, ec)],
                                 e_bufs[slot][1], sems[slot])

        start_chunk(0, 0)
        if nchunks > 1:
            start_chunk(1, 1)

        for f in range(f_tile):
            pltpu.sync_copy(hpT_hbm.at[pl.ds(fb + f, 1)], hps[f])

        def zero_body(i, carry):
            for a_v in accs:
                a_v[0, pl.ds(i * LANES, LANES)] = jnp.zeros((LANES,),
                                                            jnp.float32)
            return carry
        lax.fori_loop(0, N_PAD // LANES, zero_body, 0)

        zrow = jnp.zeros((LANES,), jnp.int32)

        def wait_chunk(slot):
            src0 = sd_hbm if packed else src_hbm
            for b in e_bufs[slot]:
                pltpu.make_async_copy(src0.at[pl.ds(0, ec)], b,
                                      sems[slot]).wait()

        def load_sd(bufs, g):
            if packed:
                sd = bufs[0][pl.ds(g * LANES, LANES)]
                return (jnp.bitwise_and(sd, jnp.int32(0x3FFF)),
                        lax.shift_right_logical(sd, jnp.int32(14)))
            return (bufs[0][pl.ds(g * LANES, LANES)],
                    bufs[1][pl.ds(g * LANES, LANES)])

        def process_chunk(bufs):
            def group_body(it, cnt):
                pairs = [load_sd(bufs, it * ng + a) for a in range(ng)]
                ss_ = [p[0] for p in pairs]
                ds_ = [p[1] for p in pairs]
                msgs = [[plsc.load_gather(hps[f], [zrow, ss_[a]])
                         for f in range(f_tile)] for a in range(ng)]
                curs = [[plsc.load_gather(acc_ref(a, f), [zrow, ds_[a]])
                         for f in range(f_tile)] for a in range(ng)]
                for a in range(ng):
                    for f in range(f_tile):
                        plsc.store_scatter(
                            acc_ref(a, f), [zrow, ds_[a]], msgs[a][f],
                            mask=msgs[a][f] > curs[a][f])
                chks = [[plsc.load_gather(acc_ref(a, f), [zrow, ds_[a]])
                         for f in range(f_tile)] for a in range(ng)]
                loses = []
                for a in range(ng):
                    lose = msgs[a][0] > chks[a][0]
                    for f in range(1, f_tile):
                        lose = jnp.logical_or(lose,
                                              msgs[a][f] > chks[a][f])
                    loses.append(lose)
                for a in range(ng):
                    nlose = plsc.all_reduce_population_count(loses[a])[0]
                    glist_s[cnt] = it * ng + a
                    cnt = cnt + (nlose > 0).astype(jnp.int32)
                return cnt
            ndup = lax.fori_loop(0, ec // (LANES * ng), group_body,
                                 jnp.int32(0))

            def dup_body(i, carry):
                g = glist_s[i]
                s, d = load_sd(bufs, g)
                for f in range(f_tile):
                    msg = plsc.load_gather(hps[f], [zrow, s])

                    def cond(cur2):
                        return jnp.any(msg > cur2)

                    def body(cur2):
                        plsc.store_scatter(accs[f], [zrow, d], msg,
                                           mask=msg > cur2)
                        return plsc.load_gather(accs[f], [zrow, d])
                    lax.while_loop(cond, body,
                                   plsc.load_gather(accs[f], [zrow, d]))
                return carry
            lax.fori_loop(0, ndup, dup_body, 0)

        def pair_body(k, carry):
            for slot in range(2):
                c = 2 * k + slot
                wait_chunk(slot)
                process_chunk(e_bufs[slot])

                @pl.when(c + 2 < nchunks)
                def _start_next(slot=slot, c=c):
                    start_chunk(slot, c + 2)
            return carry
        lax.fori_loop(0, nchunks // 2, pair_body, 0)

        if nbanks > 1:
            def merge_body(i, carry):
                for f in range(f_tile):
                    m = accs[f][0, pl.ds(i * LANES, LANES)]
                    for a in range(1, nbanks):
                        m = jnp.maximum(
                            m, accs[a * f_tile + f][0,
                                                    pl.ds(i * LANES, LANES)])
                    accs[f][0, pl.ds(i * LANES, LANES)] = m
                return carry
            lax.fori_loop(0, N_PAD // LANES, merge_body, 0)

        for f in range(f_tile):
            pltpu.sync_copy(accs[f], out_hbm.at[pl.ds(fb + f, 1)])

    return seg


def kernel(x, edge_index, W_pool1, b_pool1, W_self1, b_self1, W_neigh1,
           W_pool2, b_pool2, W_self2, b_self2, W_neigh2, W_out, b_out):
    n, d_in = x.shape
    h_dim = W_self1.shape[1]
    c_dim = W_out.shape[1]
    e = edge_index.shape[1]

    ec = 8000
    e_pad = ((e + 2 * ec - 1) // (2 * ec)) * (2 * ec)
    src = edge_index[0]
    dst = edge_index[1]
    if e_pad != e:
        src = jnp.concatenate([src, jnp.zeros((e_pad - e,), jnp.int32)])
        dst = jnp.concatenate(
            [dst, jnp.full((e_pad - e,), n, jnp.int32)])
    sd_packed = jnp.bitwise_or(src, jnp.left_shift(dst, 14))

    xp = jnp.pad(x, ((0, N_PAD - n), (0, 0)))
    grid = N_PAD // BLK

    h_pool1T = pl.pallas_call(
        _tc1_body,
        grid=(grid,),
        in_specs=[pl.BlockSpec((BLK, d_in), lambda i: (i, 0)),
                  _full((d_in, d_in)),
                  _full((d_in, 1))],
        out_specs=pl.BlockSpec((d_in, BLK), lambda i: (0, i)),
        out_shape=jax.ShapeDtypeStruct((d_in, N_PAD), jnp.float32),
    )(xp, W_pool1, b_pool1.reshape(d_in, 1))

    seg1 = _make_segmax(d_in, d_in // (NC * NS), e_pad, ec, ng=4, nbanks=1,
                        packed=True)
    h_neigh1T = seg1(h_pool1T, sd_packed)

    hT, h_pool2T = pl.pallas_call(
        _tc2_body,
        grid=(grid,),
        in_specs=[pl.BlockSpec((BLK, d_in), lambda i: (i, 0)),
                  pl.BlockSpec((d_in, BLK), lambda i: (0, i)),
                  _full((d_in, h_dim)),
                  _full((h_dim, 1)),
                  _full((d_in, h_dim)),
                  _full((h_dim, h_dim)),
                  _full((h_dim, 1))],
        out_specs=[pl.BlockSpec((h_dim, BLK), lambda i: (0, i)),
                   pl.BlockSpec((h_dim, BLK), lambda i: (0, i))],
        out_shape=[jax.ShapeDtypeStruct((h_dim, N_PAD), jnp.float32),
                   jax.ShapeDtypeStruct((h_dim, N_PAD), jnp.float32)],
    )(xp, h_neigh1T, W_self1, b_self1.reshape(h_dim, 1), W_neigh1,
      W_pool2, b_pool2.reshape(h_dim, 1))

    seg2 = _make_segmax(h_dim, h_dim // (NC * NS), e_pad, ec, ng=10,
                        nbanks=10, packed=True)
    h_neigh2T = seg2(h_pool2T, sd_packed)

    outT = pl.pallas_call(
        _tc3_body,
        grid=(grid,),
        in_specs=[pl.BlockSpec((h_dim, BLK), lambda i: (0, i)),
                  pl.BlockSpec((h_dim, BLK), lambda i: (0, i)),
                  _full((h_dim, h_dim)),
                  _full((h_dim, 1)),
                  _full((h_dim, h_dim)),
                  _full((h_dim, c_dim)),
                  _full((c_dim, 1))],
        out_specs=pl.BlockSpec((c_dim, BLK), lambda i: (0, i)),
        out_shape=jax.ShapeDtypeStruct((c_dim, N_PAD), jnp.float32),
    )(hT, h_neigh2T, W_self2, b_self2.reshape(h_dim, 1), W_neigh2,
      W_out, b_out.reshape(c_dim, 1))

    return outT[:, :n].T


# speedup vs baseline: 1.2020x; 1.0005x over previous
import functools

import jax
import jax.numpy as jnp
from jax import lax
from jax.experimental import pallas as pl
from jax.experimental.pallas import tpu as pltpu
from jax.experimental.pallas import tpu_sc as plsc

N_PAD = 10240
BLK = 1024
NC = 2
NS = 16
LANES = 16

_PREC = jax.lax.Precision.DEFAULT


def _tc1_body(x_ref, wp_ref, bp_ref, out_ref):
    acc = lax.dot_general(wp_ref[...], x_ref[...], (((0,), (1,)), ((), ())),
                          precision=_PREC, preferred_element_type=jnp.float32)
    out_ref[...] = jnp.maximum(acc + bp_ref[...], 0.0)


def _tc2_body(x_ref, hn_ref, ws_ref, bs_ref, wn_ref, wp2_ref, bp2_ref,
              h_ref, hp2_ref):
    rst = lax.dot_general(ws_ref[...], x_ref[...], (((0,), (1,)), ((), ())),
                          precision=_PREC, preferred_element_type=jnp.float32)
    rst = rst + bs_ref[...]
    rst = rst + lax.dot_general(wn_ref[...], hn_ref[...],
                                (((0,), (0,)), ((), ())),
                                precision=_PREC,
                                preferred_element_type=jnp.float32)
    h = jnp.maximum(rst, 0.0)
    h_ref[...] = h
    hp2 = lax.dot_general(wp2_ref[...], h, (((0,), (0,)), ((), ())),
                          precision=_PREC, preferred_element_type=jnp.float32)
    hp2_ref[...] = jnp.maximum(hp2 + bp2_ref[...], 0.0)


def _tc3_body(h_ref, hn2_ref, ws2_ref, bs2_ref, wn2_ref, wo_ref, bo_ref,
              out_ref):
    rst = lax.dot_general(ws2_ref[...], h_ref[...], (((0,), (0,)), ((), ())),
                          precision=_PREC, preferred_element_type=jnp.float32)
    rst = rst + bs2_ref[...]
    rst = rst + lax.dot_general(wn2_ref[...], hn2_ref[...],
                                (((0,), (0,)), ((), ())),
                                precision=_PREC,
                                preferred_element_type=jnp.float32)
    h2 = jnp.maximum(rst, 0.0)
    out = lax.dot_general(wo_ref[...], h2, (((0,), (0,)), ((), ())),
                          precision=_PREC, preferred_element_type=jnp.float32)
    out_ref[...] = out + bo_ref[...]


def _full(shape):
    return pl.BlockSpec(shape, lambda i: (0, 0))


def _make_segmax(f_total, f_tile, e_pad, ec, ng, nbanks, packed):
    nchunks = e_pad // ec
    mesh = plsc.VectorSubcoreMesh(core_axis_name="c", subcore_axis_name="s",
                                  num_cores=NC, num_subcores=NS)

    scratch = (
        [pltpu.VMEM((1, N_PAD), jnp.float32)] * f_tile
        + [pltpu.VMEM((1, N_PAD), jnp.float32)] * (f_tile * nbanks)
        + [pltpu.VMEM((ec,), jnp.int32)] * (2 if packed else 4)
        + [pltpu.SMEM((ec // LANES,), jnp.int32),
           pltpu.SemaphoreType.DMA, pltpu.SemaphoreType.DMA]
    )

    @functools.partial(
        pl.kernel,
        out_type=jax.ShapeDtypeStruct((f_total, N_PAD), jnp.float32),
        mesh=mesh,
        compiler_params=pltpu.CompilerParams(needs_layout_passes=False),
        scratch_types=scratch,
    )
    def seg(hpT_hbm, *args):
        if packed:
            (sd_hbm, out_hbm), scr = args[:2], args[2:]
        else:
            (src_hbm, dst_hbm, out_hbm), scr = args[:3], args[3:]
        hps = scr[:f_tile]
        accs = scr[f_tile:f_tile + f_tile * nbanks]
        rest = scr[f_tile + f_tile * nbanks:]
        if packed:
            e_bufs = ((rest[0],), (rest[1],))
            glist_s, sem0, sem1 = rest[2:]
        else:
            e_bufs = ((rest[0], rest[2]), (rest[1], rest[3]))
            glist_s, sem0, sem1 = rest[4:]
        sems = (sem0, sem1)

        def acc_ref(a, f):
            return accs[(a % nbanks) * f_tile + f]

        wid = lax.axis_index("s") * NC + lax.axis_index("c")
        fb = wid * f_tile

        def start_chunk(slot, cidx):
            if packed:
                pltpu.async_copy(sd_hbm.at[pl.ds(cidx * ec, ec)],
                                 e_bufs[slot][0], sems[slot])
            else:
                pltpu.async_copy(src_hbm.at[pl.ds(cidx * ec, ec)],
                                 e_bufs[slot][0], sems[slot])
                pltpu.async_copy(dst_hbm.at[pl.ds(cidx * ec, ec)],
                                 e_bufs[slot][1], sems[slot])

        start_chunk(0, 0)
        if nchunks > 1:
            start_chunk(1, 1)

        for f in range(f_tile):
            pltpu.sync_copy(hpT_hbm.at[pl.ds(fb + f, 1)], hps[f])

        def zero_body(i, carry):
            for a_v in accs:
                a_v[0, pl.ds(i * LANES, LANES)] = jnp.zeros((LANES,),
                                                            jnp.float32)
            return carry
        lax.fori_loop(0, N_PAD // LANES, zero_body, 0)

        zrow = jnp.zeros((LANES,), jnp.int32)

        def wait_chunk(slot):
            src0 = sd_hbm if packed else src_hbm
            for b in e_bufs[slot]:
                pltpu.make_async_copy(src0.at[pl.ds(0, ec)], b,
                                      sems[slot]).wait()

        def load_sd(bufs, g):
            if packed:
                sd = bufs[0][pl.ds(g * LANES, LANES)]
                return (jnp.bitwise_and(sd, jnp.int32(0x3FFF)),
                        lax.shift_right_logical(sd, jnp.int32(14)))
            return (bufs[0][pl.ds(g * LANES, LANES)],
                    bufs[1][pl.ds(g * LANES, LANES)])

        def process_chunk(bufs):
            def group_body(it, cnt):
                pairs = [load_sd(bufs, it * ng + a) for a in range(ng)]
                ss_ = [p[0] for p in pairs]
                ds_ = [p[1] for p in pairs]
                msgs = [[plsc.load_gather(hps[f], [zrow, ss_[a]])
                         for f in range(f_tile)] for a in range(ng)]
                curs = [[plsc.load_gather(acc_ref(a, f), [zrow, ds_[a]])
                         for f in range(f_tile)] for a in range(ng)]
                for a in range(ng):
                    for f in range(f_tile):
                        plsc.store_scatter(
                            acc_ref(a, f), [zrow, ds_[a]], msgs[a][f],
                            mask=msgs[a][f] > curs[a][f])
                chks = [[plsc.load_gather(acc_ref(a, f), [zrow, ds_[a]])
                         for f in range(f_tile)] for a in range(ng)]
                loses = []
                for a in range(ng):
                    lose = msgs[a][0] > chks[a][0]
                    for f in range(1, f_tile):
                        lose = jnp.logical_or(lose,
                                              msgs[a][f] > chks[a][f])
                    loses.append(lose)
                for a in range(ng):
                    nlose = plsc.all_reduce_population_count(loses[a])[0]
                    glist_s[cnt] = it * ng + a
                    cnt = cnt + (nlose > 0).astype(jnp.int32)
                return cnt
            ndup = lax.fori_loop(0, ec // (LANES * ng), group_body,
                                 jnp.int32(0))

            def dup_body(i, carry):
                g = glist_s[i]
                s, d = load_sd(bufs, g)
                for f in range(f_tile):
                    msg = plsc.load_gather(hps[f], [zrow, s])

                    def cond(cur2):
                        return jnp.any(msg > cur2)

                    def body(cur2):
                        plsc.store_scatter(accs[f], [zrow, d], msg,
                                           mask=msg > cur2)
                        return plsc.load_gather(accs[f], [zrow, d])
                    lax.while_loop(cond, body,
                                   plsc.load_gather(accs[f], [zrow, d]))
                return carry
            lax.fori_loop(0, ndup, dup_body, 0)

        def pair_body(k, carry):
            for slot in range(2):
                c = 2 * k + slot
                wait_chunk(slot)
                process_chunk(e_bufs[slot])

                @pl.when(c + 2 < nchunks)
                def _start_next(slot=slot, c=c):
                    start_chunk(slot, c + 2)
            return carry
        lax.fori_loop(0, nchunks // 2, pair_body, 0)

        if nbanks > 1:
            def merge_body(i, carry):
                for f in range(f_tile):
                    m = accs[f][0, pl.ds(i * LANES, LANES)]
                    for a in range(1, nbanks):
                        m = jnp.maximum(
                            m, accs[a * f_tile + f][0,
                                                    pl.ds(i * LANES, LANES)])
                    accs[f][0, pl.ds(i * LANES, LANES)] = m
                return carry
            lax.fori_loop(0, N_PAD // LANES, merge_body, 0)

        for f in range(f_tile):
            pltpu.sync_copy(accs[f], out_hbm.at[pl.ds(fb + f, 1)])

    return seg


def kernel(x, edge_index, W_pool1, b_pool1, W_self1, b_self1, W_neigh1,
           W_pool2, b_pool2, W_self2, b_self2, W_neigh2, W_out, b_out):
    n, d_in = x.shape
    h_dim = W_self1.shape[1]
    c_dim = W_out.shape[1]
    e = edge_index.shape[1]

    ec = 8000
    e_pad = ((e + 2 * ec - 1) // (2 * ec)) * (2 * ec)
    src = edge_index[0]
    dst = edge_index[1]
    if e_pad != e:
        src = jnp.concatenate([src, jnp.zeros((e_pad - e,), jnp.int32)])
        dst = jnp.concatenate(
            [dst, jnp.full((e_pad - e,), n, jnp.int32)])
    sd_packed = jnp.bitwise_or(src, jnp.left_shift(dst, 14))

    xp = jnp.pad(x, ((0, N_PAD - n), (0, 0)))
    grid = N_PAD // BLK

    h_pool1T = pl.pallas_call(
        _tc1_body,
        grid=(grid,),
        in_specs=[pl.BlockSpec((BLK, d_in), lambda i: (i, 0)),
                  _full((d_in, d_in)),
                  _full((d_in, 1))],
        out_specs=pl.BlockSpec((d_in, BLK), lambda i: (0, i)),
        out_shape=jax.ShapeDtypeStruct((d_in, N_PAD), jnp.float32),
    )(xp, W_pool1, b_pool1.reshape(d_in, 1))

    seg1 = _make_segmax(d_in, d_in // (NC * NS), e_pad, ec, ng=4, nbanks=1,
                        packed=True)
    h_neigh1T = seg1(h_pool1T, sd_packed)

    hT, h_pool2T = pl.pallas_call(
        _tc2_body,
        grid=(grid,),
        in_specs=[pl.BlockSpec((BLK, d_in), lambda i: (i, 0)),
                  pl.BlockSpec((d_in, BLK), lambda i: (0, i)),
                  _full((d_in, h_dim)),
                  _full((h_dim, 1)),
                  _full((d_in, h_dim)),
                  _full((h_dim, h_dim)),
                  _full((h_dim, 1))],
        out_specs=[pl.BlockSpec((h_dim, BLK), lambda i: (0, i)),
                   pl.BlockSpec((h_dim, BLK), lambda i: (0, i))],
        out_shape=[jax.ShapeDtypeStruct((h_dim, N_PAD), jnp.float32),
                   jax.ShapeDtypeStruct((h_dim, N_PAD), jnp.float32)],
    )(xp, h_neigh1T, W_self1, b_self1.reshape(h_dim, 1), W_neigh1,
      W_pool2, b_pool2.reshape(h_dim, 1))

    seg2 = _make_segmax(h_dim, h_dim // (NC * NS), e_pad, ec, ng=10,
                        nbanks=10, packed=True)
    h_neigh2T = seg2(h_pool2T, sd_packed)

    outT = pl.pallas_call(
        _tc3_body,
        grid=(grid,),
        in_specs=[pl.BlockSpec((h_dim, BLK), lambda i: (0, i)),
                  pl.BlockSpec((h_dim, BLK), lambda i: (0, i)),
                  _full((h_dim, h_dim)),
                  _full((h_dim, 1)),
                  _full((h_dim, h_dim)),
                  _full((h_dim, c_dim)),
                  _full((c_dim, 1))],
        out_specs=pl.BlockSpec((c_dim, BLK), lambda i: (0, i)),
        out_shape=jax.ShapeDtypeStruct((c_dim, N_PAD), jnp.float32),
    )(hT, h_neigh2T, W_self2, b_self2.reshape(h_dim, 1), W_neigh2,
      W_out, b_out.reshape(c_dim, 1))

    return outT[:, :n].T
